# baseline (speedup 1.0000x reference)
import functools

import jax
import jax.numpy as jnp
from jax import lax
from jax.experimental import pallas as pl
from jax.experimental.pallas import tpu as pltpu
from jax.experimental.pallas import tpu_sc as plsc

B = 16384
D = 1024
NW = 32
ROWS_PER_W = B // NW
RBLK = 16
NBLK = ROWS_PER_W // RBLK


def _sc_permute_x(x, perm):
    mesh = plsc.VectorSubcoreMesh(core_axis_name="c", subcore_axis_name="s")

    @functools.partial(
        pl.kernel,
        mesh=mesh,
        compiler_params=pltpu.CompilerParams(needs_layout_passes=False),
        out_type=jax.ShapeDtypeStruct((B, D), jnp.float32),
        scratch_types=[
            pltpu.VMEM((D,), jnp.int32),
            pltpu.VMEM((RBLK, D), jnp.float32),
            pltpu.VMEM((RBLK, D), jnp.float32),
            pltpu.VMEM((RBLK, D), jnp.float32),
            pltpu.VMEM((RBLK, D), jnp.float32),
            pltpu.SemaphoreType.DMA,
            pltpu.SemaphoreType.DMA,
            pltpu.SemaphoreType.DMA,
            pltpu.SemaphoreType.DMA,
        ],
    )
    def k(x_hbm, perm_hbm, xo_hbm,
          perm_v, xin0, xin1, xout0, xout1, si0, si1, so0, so1):
        wid = lax.axis_index("s") * 2 + lax.axis_index("c")
        base = wid * ROWS_PER_W

        xin = (xin0, xin1)
        xout = (xout0, xout1)
        si = (si0, si1)
        so = (so0, so1)
        rsp = [jnp.full((16,), r, jnp.int32) for r in range(RBLK)]

        H2 = RBLK // 2

        def start_in(bi, p):
            for h in (0, 1):
                pltpu.make_async_copy(
                    x_hbm.at[pl.ds(base + bi * RBLK + h * H2, H2)],
                    xin[p].at[pl.ds(h * H2, H2)], si[p]
                ).start()

        def wait_in(p):
            for h in (0, 1):
                pltpu.make_async_copy(
                    x_hbm.at[pl.ds(base, H2)],
                    xin[p].at[pl.ds(h * H2, H2)], si[p]
                ).wait()

        def start_out(bi, p):
            for h in (0, 1):
                pltpu.make_async_copy(
                    xout[p].at[pl.ds(h * H2, H2)],
                    xo_hbm.at[pl.ds(base + bi * RBLK + h * H2, H2)], so[p]
                ).start()

        def wait_out(p):
            for h in (0, 1):
                pltpu.make_async_copy(
                    xout[p].at[pl.ds(h * H2, H2)],
                    xo_hbm.at[pl.ds(base, H2)], so[p]
                ).wait()

        def compute(p):
            xin_p = xin[p]
            xout_p = xout[p]
            H = RBLK // 2

            def gather_half(colv, g0):
                return tuple(
                    plsc.load_gather(xin_p, [rsp[g0 + u], colv])
                    for u in range(H)
                )

            def store_half(vals, obase, g0):
                for u in range(H):
                    xout_p[g0 + u, pl.ds(obase, 16)] = vals[u]

            colv0 = perm_v[pl.ds(0, 16)]
            lo0 = gather_half(colv0, 0)
            store_half(lo0, 0, 0)
            hi0 = gather_half(colv0, H)

            def x_outer(j, carry):
                obase = j * 16
                colv = perm_v[pl.ds(obase, 16)]
                lo = gather_half(colv, 0)
                store_half(carry, obase - 16, H)
                hi = gather_half(colv, H)
                store_half(lo, obase, 0)
                return hi

            last = lax.fori_loop(1, D // 16, x_outer, hi0, unroll=False)
            store_half(last, D - 16, H)

        start_in(0, 0)
        start_in(1, 1)
        pltpu.sync_copy(perm_hbm, perm_v)

        def body(hi, _):
            for p in (0, 1):
                bi = hi * 2 + p

                wait_in(p)

                @pl.when(bi >= 2)
                def _():
                    wait_out(p)

                compute(p)

                @pl.when(bi + 2 < NBLK)
                def _():
                    start_in(bi + 2, p)

                start_out(bi, p)
            return 0

        lax.fori_loop(0, NBLK // 2, body, 0, unroll=False)
        wait_out(0)
        wait_out(1)

    return k(x, perm)


def _tc_build_p(perm):
    def build(perm_ref, p_ref):
        col = lax.broadcasted_iota(jnp.int32, (D, D), 0)
        pj = perm_ref[...]
        p_ref[...] = (pj[None, :] == col).astype(jnp.int8)

    return pl.pallas_call(
        build,
        out_shape=jax.ShapeDtypeStruct((D, D), jnp.int8),
    )(perm)


def _tc_permute_mask(mask_i8, p_mat):
    rb = 2048

    def mm(m_ref, p_ref, o_ref):
        acc = jnp.dot(m_ref[...], p_ref[...],
                      preferred_element_type=jnp.int32)
        o_ref[...] = acc.astype(jnp.int8)

    return pl.pallas_call(
        mm,
        grid=(B // rb,),
        in_specs=[
            pl.BlockSpec((rb, D), lambda i: (i, 0)),
            pl.BlockSpec((D, D), lambda i: (0, 0)),
        ],
        out_specs=pl.BlockSpec((rb, D), lambda i: (i, 0)),
        out_shape=jax.ShapeDtypeStruct((B, D), jnp.int8),
    )(mask_i8, p_mat)


def kernel(x, observed_mask, perm, inv_perm):
    del inv_perm
    xo = _sc_permute_x(x, perm)
    p_mat = _tc_build_p(perm)
    mo_i8 = _tc_permute_mask(observed_mask.astype(jnp.int8), p_mat)
    return (xo, mo_i8 != 0)


# speedup vs baseline: 1.0036x; 1.0036x over previous
import functools

import jax
import jax.numpy as jnp
from jax import lax
from jax.experimental import pallas as pl
from jax.experimental.pallas import tpu as pltpu
from jax.experimental.pallas import tpu_sc as plsc

B = 16384
D = 1024
NW = 32
ROWS_PER_W = B // NW
RBLK = 16
NBLK = ROWS_PER_W // RBLK


def _sc_permute_x(x, perm):
    mesh = plsc.VectorSubcoreMesh(core_axis_name="c", subcore_axis_name="s")

    @functools.partial(
        pl.kernel,
        mesh=mesh,
        compiler_params=pltpu.CompilerParams(needs_layout_passes=False),
        out_type=jax.ShapeDtypeStruct((B, D), jnp.float32),
        scratch_types=[
            pltpu.VMEM((D,), jnp.int32),
            pltpu.VMEM((RBLK, D), jnp.float32),
            pltpu.VMEM((RBLK, D), jnp.float32),
            pltpu.VMEM((RBLK, D), jnp.float32),
            pltpu.VMEM((RBLK, D), jnp.float32),
            pltpu.SemaphoreType.DMA,
            pltpu.SemaphoreType.DMA,
            pltpu.SemaphoreType.DMA,
            pltpu.SemaphoreType.DMA,
        ],
    )
    def k(x_hbm, perm_hbm, xo_hbm,
          perm_v, xin0, xin1, xout0, xout1, si0, si1, so0, so1):
        wid = lax.axis_index("s") * 2 + lax.axis_index("c")
        base = wid * ROWS_PER_W

        xin = (xin0, xin1)
        xout = (xout0, xout1)
        si = (si0, si1)
        so = (so0, so1)
        rsp = [jnp.full((16,), r, jnp.int32) for r in range(RBLK)]

        def start_in(bi, p):
            pltpu.make_async_copy(
                x_hbm.at[pl.ds(base + bi * RBLK, RBLK)], xin[p], si[p]
            ).start()

        def wait_in(p):
            pltpu.make_async_copy(
                x_hbm.at[pl.ds(base, RBLK)], xin[p], si[p]
            ).wait()

        def start_out(bi, p):
            pltpu.make_async_copy(
                xout[p], xo_hbm.at[pl.ds(base + bi * RBLK, RBLK)], so[p]
            ).start()

        def wait_out(p):
            pltpu.make_async_copy(
                xout[p], xo_hbm.at[pl.ds(base, RBLK)], so[p]
            ).wait()

        def compute(p):
            xin_p = xin[p]
            xout_p = xout[p]
            H = RBLK // 2

            def gather_half(colv, g0):
                return tuple(
                    plsc.load_gather(xin_p, [rsp[g0 + u], colv])
                    for u in range(H)
                )

            def store_half(vals, obase, g0):
                for u in range(H):
                    xout_p[g0 + u, pl.ds(obase, 16)] = vals[u]

            colv0 = perm_v[pl.ds(0, 16)]
            lo0 = gather_half(colv0, 0)
            store_half(lo0, 0, 0)
            hi0 = gather_half(colv0, H)

            def x_outer(j, carry):
                obase = j * 16
                colv = perm_v[pl.ds(obase, 16)]
                lo = gather_half(colv, 0)
                store_half(carry, obase - 16, H)
                hi = gather_half(colv, H)
                store_half(lo, obase, 0)
                return hi

            last = lax.fori_loop(1, D // 16, x_outer, hi0, unroll=False)
            store_half(last, D - 16, H)

        start_in(0, 0)
        start_in(1, 1)
        pltpu.sync_copy(perm_hbm, perm_v)

        def body(hi, _):
            for p in (0, 1):
                bi = hi * 2 + p

                wait_in(p)

                @pl.when(bi >= 2)
                def _():
                    wait_out(p)

                compute(p)

                @pl.when(bi + 2 < NBLK)
                def _():
                    start_in(bi + 2, p)

                start_out(bi, p)
            return 0

        lax.fori_loop(0, NBLK // 2, body, 0, unroll=False)
        wait_out(0)
        wait_out(1)

    return k(x, perm)


def _tc_build_p(perm):
    def build(perm_ref, p_ref):
        col = lax.broadcasted_iota(jnp.int32, (D, D), 0)
        pj = perm_ref[...]
        p_ref[...] = (pj[None, :] == col).astype(jnp.int8)

    return pl.pallas_call(
        build,
        out_shape=jax.ShapeDtypeStruct((D, D), jnp.int8),
    )(perm)


def _tc_permute_mask(mask_i8, p_mat):
    rb = 2048

    def mm(m_ref, p_ref, o_ref):
        acc = jnp.dot(m_ref[...], p_ref[...],
                      preferred_element_type=jnp.int32)
        o_ref[...] = acc.astype(jnp.int8)

    return pl.pallas_call(
        mm,
        grid=(B // rb,),
        in_specs=[
            pl.BlockSpec((rb, D), lambda i: (i, 0)),
            pl.BlockSpec((D, D), lambda i: (0, 0)),
        ],
        out_specs=pl.BlockSpec((rb, D), lambda i: (i, 0)),
        out_shape=jax.ShapeDtypeStruct((B, D), jnp.int8),
    )(mask_i8, p_mat)


def kernel(x, observed_mask, perm, inv_perm):
    del inv_perm
    xo = _sc_permute_x(x, perm)
    p_mat = _tc_build_p(perm)
    mo_i8 = _tc_permute_mask(observed_mask.astype(jnp.int8), p_mat)
    return (xo, mo_i8 != 0)
